# Initial kernel scaffold; baseline (speedup 1.0000x reference)
import functools

import jax
import jax.numpy as jnp
from jax.experimental import pallas as pl
from jax.experimental.pallas import tpu as pltpu


def _moe_body(x_ref, gw_ref, gb_ref, w1_ref, b1_ref, w2_ref, b2_ref,
              out_ref, xb_ref, coef_ref, *, n_shared, n_routed, tile):
    e = pl.program_id(0)
    t = pl.program_id(1)
    rows = pl.ds(t * tile, tile)

    @pl.when(e == 0)
    def _gating():
        x = x_ref[...]
        xb_ref[rows, :] = x.astype(jnp.bfloat16)
        logits = jnp.dot(x, gw_ref[...], preferred_element_type=jnp.float32,
                         precision=jax.lax.Precision.HIGHEST) + gb_ref[...]
        m = jnp.max(logits, axis=-1, keepdims=True)
        ex = jnp.exp(logits - m)
        w = ex / jnp.sum(ex, axis=-1, keepdims=True)
        iota = jax.lax.broadcasted_iota(jnp.int32, w.shape, 1)
        m1 = jnp.max(w, axis=-1, keepdims=True)
        i1 = jnp.min(jnp.where(w == m1, iota, n_routed), axis=-1, keepdims=True)
        sel1 = iota == i1
        w_rest = jnp.where(sel1, -1.0, w)
        m2 = jnp.max(w_rest, axis=-1, keepdims=True)
        i2 = jnp.min(jnp.where(w_rest == m2, iota, n_routed), axis=-1,
                     keepdims=True)
        sel2 = iota == i2
        coef_ref[rows, :] = jnp.where(sel1 | sel2, w, 0.0)

    xb = xb_ref[rows, :]
    h = jnp.dot(xb, w1_ref[0], preferred_element_type=jnp.float32)
    h = jnp.maximum(h + b1_ref[0], 0.0).astype(jnp.bfloat16)
    y = jnp.dot(h, w2_ref[0], preferred_element_type=jnp.float32) + b2_ref[0]

    ec = coef_ref[rows, :]
    lane = jax.lax.broadcasted_iota(jnp.int32, ec.shape, 1)
    routed_col = jnp.sum(jnp.where(lane == (e - n_shared), ec, 0.0), axis=-1,
                         keepdims=True)
    coef = jnp.where(e < n_shared, 1.0 / n_shared, routed_col)
    y = y * coef

    @pl.when(e == 0)
    def _init():
        out_ref[rows, :] = y

    @pl.when(e > 0)
    def _accum():
        out_ref[rows, :] += y


def _moe_call(x2d, gate_W, gate_b2d, W1all, b1all, W2all, b2all,
              n_shared, n_routed, tile, interpret=False):
    S, D = x2d.shape
    NE = n_shared + n_routed
    F = W1all.shape[-1]
    O = W2all.shape[-1]
    nt = S // tile
    body = functools.partial(_moe_body, n_shared=n_shared, n_routed=n_routed,
                             tile=tile)
    return pl.pallas_call(
        body,
        grid=(NE, nt),
        in_specs=[
            pl.BlockSpec((tile, D), lambda e, t: (jnp.where(e == 0, t, 0), 0)),
            pl.BlockSpec((D, n_routed), lambda e, t: (0, 0)),
            pl.BlockSpec((1, n_routed), lambda e, t: (0, 0)),
            pl.BlockSpec((1, D, F), lambda e, t: (e, 0, 0)),
            pl.BlockSpec((1, 1, F), lambda e, t: (e, 0, 0)),
            pl.BlockSpec((1, F, O), lambda e, t: (e, 0, 0)),
            pl.BlockSpec((1, 1, O), lambda e, t: (e, 0, 0)),
        ],
        out_specs=pl.BlockSpec((S, O), lambda e, t: (0, 0)),
        out_shape=jax.ShapeDtypeStruct((S, O), jnp.float32),
        scratch_shapes=[
            pltpu.VMEM((S, D), jnp.bfloat16),
            pltpu.VMEM((S, n_routed), jnp.float32),
        ],
        interpret=interpret,
    )(x2d, gate_W, gate_b2d, W1all, b1all, W2all, b2all)


def kernel(x, gate_W, gate_b, shared_W1, shared_b1, shared_W2, shared_b2,
           routed_W1, routed_b1, routed_W2, routed_b2):
    B, S, D = x.shape
    n_shared = shared_W1.shape[0]
    n_routed = routed_W1.shape[0]
    F = shared_W1.shape[-1]
    O = shared_W2.shape[-1]

    x2d = x.reshape(S, D)
    W1all = jnp.concatenate([shared_W1, routed_W1], axis=0).astype(jnp.bfloat16)
    W2all = jnp.concatenate([shared_W2, routed_W2], axis=0).astype(jnp.bfloat16)
    b1all = jnp.concatenate([shared_b1, routed_b1], axis=0).reshape(
        n_shared + n_routed, 1, F)
    b2all = jnp.concatenate([shared_b2, routed_b2], axis=0).reshape(
        n_shared + n_routed, 1, O)
    gate_b2d = gate_b.reshape(1, n_routed)

    out = _moe_call(x2d, gate_W, gate_b2d, W1all, b1all, W2all, b2all,
                    n_shared, n_routed, tile=512)
    return out.reshape(B, S, O)


# speedup vs baseline: 1.2258x; 1.2258x over previous
import functools

import jax
import jax.numpy as jnp
from jax.experimental import pallas as pl
from jax.experimental.pallas import tpu as pltpu


def _moe_body(x_ref, gw_ref, gb_ref, w1_ref, b1_ref, w2_ref, b2_ref,
              out_ref, xb_ref, coef_ref, *, n_shared, n_routed, tile):
    e = pl.program_id(0)
    t = pl.program_id(1)
    rows = pl.ds(t * tile, tile)

    @pl.when(e == 0)
    def _gating():
        x = x_ref[...]
        xh = x.astype(jnp.bfloat16)
        xb_ref[rows, :] = xh
        gh = gw_ref[...].astype(jnp.bfloat16)
        logits = (jnp.dot(xh, gh, preferred_element_type=jnp.float32)
                  + gb_ref[...])
        m = jnp.max(logits, axis=-1, keepdims=True)
        ex = jnp.exp(logits - m)
        w = ex / jnp.sum(ex, axis=-1, keepdims=True)
        iota = jax.lax.broadcasted_iota(jnp.int32, w.shape, 1)
        m1 = jnp.max(w, axis=-1, keepdims=True)
        i1 = jnp.min(jnp.where(w == m1, iota, n_routed), axis=-1, keepdims=True)
        sel1 = iota == i1
        w_rest = jnp.where(sel1, -1.0, w)
        m2 = jnp.max(w_rest, axis=-1, keepdims=True)
        i2 = jnp.min(jnp.where(w_rest == m2, iota, n_routed), axis=-1,
                     keepdims=True)
        sel2 = iota == i2
        coef_ref[rows, :] = jnp.where(sel1 | sel2, w, 0.0)

    xb = xb_ref[rows, :]
    h = jnp.dot(xb, w1_ref[0], preferred_element_type=jnp.float32)
    h = jnp.maximum(h + b1_ref[0], 0.0).astype(jnp.bfloat16)
    y = jnp.dot(h, w2_ref[0], preferred_element_type=jnp.float32) + b2_ref[0]

    ec = coef_ref[rows, :]
    lane = jax.lax.broadcasted_iota(jnp.int32, ec.shape, 1)
    routed_col = jnp.sum(jnp.where(lane == (e - n_shared), ec, 0.0), axis=-1,
                         keepdims=True)
    coef = jnp.where(e < n_shared, 1.0 / n_shared, routed_col)
    y = y * coef

    @pl.when(e == 0)
    def _init():
        out_ref[rows, :] = y

    @pl.when(e > 0)
    def _accum():
        out_ref[rows, :] += y


def _moe_call(x2d, gate_W, gate_b2d, W1all, b1all, W2all, b2all,
              n_shared, n_routed, tile, interpret=False):
    S, D = x2d.shape
    NE = n_shared + n_routed
    F = W1all.shape[-1]
    O = W2all.shape[-1]
    nt = S // tile
    body = functools.partial(_moe_body, n_shared=n_shared, n_routed=n_routed,
                             tile=tile)
    return pl.pallas_call(
        body,
        grid=(NE, nt),
        in_specs=[
            pl.BlockSpec((tile, D), lambda e, t: (jnp.where(e == 0, t, 0), 0)),
            pl.BlockSpec((D, n_routed), lambda e, t: (0, 0)),
            pl.BlockSpec((1, n_routed), lambda e, t: (0, 0)),
            pl.BlockSpec((1, D, F), lambda e, t: (e, 0, 0)),
            pl.BlockSpec((1, 1, F), lambda e, t: (e, 0, 0)),
            pl.BlockSpec((1, F, O), lambda e, t: (e, 0, 0)),
            pl.BlockSpec((1, 1, O), lambda e, t: (e, 0, 0)),
        ],
        out_specs=pl.BlockSpec((S, O), lambda e, t: (0, 0)),
        out_shape=jax.ShapeDtypeStruct((S, O), jnp.float32),
        scratch_shapes=[
            pltpu.VMEM((S, D), jnp.bfloat16),
            pltpu.VMEM((S, n_routed), jnp.float32),
        ],
        interpret=interpret,
    )(x2d, gate_W, gate_b2d, W1all, b1all, W2all, b2all)


def kernel(x, gate_W, gate_b, shared_W1, shared_b1, shared_W2, shared_b2,
           routed_W1, routed_b1, routed_W2, routed_b2):
    B, S, D = x.shape
    n_shared = shared_W1.shape[0]
    n_routed = routed_W1.shape[0]
    F = shared_W1.shape[-1]
    O = shared_W2.shape[-1]

    x2d = x.reshape(S, D)
    W1all = jnp.concatenate([shared_W1, routed_W1], axis=0).astype(jnp.bfloat16)
    W2all = jnp.concatenate([shared_W2, routed_W2], axis=0).astype(jnp.bfloat16)
    b1all = jnp.concatenate([shared_b1, routed_b1], axis=0).reshape(
        n_shared + n_routed, 1, F)
    b2all = jnp.concatenate([shared_b2, routed_b2], axis=0).reshape(
        n_shared + n_routed, 1, O)
    gate_b2d = gate_b.reshape(1, n_routed)

    out = _moe_call(x2d, gate_W, gate_b2d, W1all, b1all, W2all, b2all,
                    n_shared, n_routed, tile=512)
    return out.reshape(B, S, O)
